# baseline (device time: 12508 ns/iter reference)
import jax
import jax.numpy as jnp
from jax import lax
from jax.experimental import pallas as pl
from jax.experimental.pallas import tpu as pltpu

BLK = 128


def kernel(x):
    m, n = x.shape
    n_blk = m // BLK

    def body(x_ref, out_ref):
        r = lax.broadcasted_iota(jnp.int32, (BLK, BLK), 0)
        c = lax.broadcasted_iota(jnp.int32, (BLK, BLK), 1)
        tri = (r >= c).astype(jnp.bfloat16)

        total = jnp.sum(x_ref[:, :], axis=0, keepdims=True)
        off = total * 0.0
        for g in range(n_blk):
            blk = x_ref[pl.ds(g * BLK, BLK), :].astype(jnp.bfloat16)
            cs = jax.lax.dot(tri, blk, preferred_element_type=jnp.float32)
            out_ref[pl.ds(g * BLK, BLK), :] = (cs + off).astype(jnp.bfloat16)
            off = off + cs[BLK - 1 : BLK, :]

    return pl.pallas_call(
        body,
        out_shape=jax.ShapeDtypeStruct((m, n), jnp.bfloat16),
        in_specs=[pl.BlockSpec(memory_space=pltpu.VMEM)],
        out_specs=pl.BlockSpec(memory_space=pltpu.VMEM),
    )(x)
